# Initial kernel scaffold; baseline (speedup 1.0000x reference)
import jax
import jax.numpy as jnp
from jax.experimental import pallas as pl


def kernel(x, W):
    raise NotImplementedError("write your pallas kernel here")


# speedup vs baseline: 3.1368x; 3.1368x over previous
import functools

import jax
import jax.numpy as jnp
from jax.experimental import pallas as pl
from jax.experimental.pallas import tpu as pltpu


def _fused_body(x_ref, w_ref, scores_ref, idx_ref, wts_ref, *, K):
    T = x_ref.shape[0]
    E = w_ref.shape[0]
    logits = jax.lax.dot_general(
        x_ref[...], w_ref[...],
        dimension_numbers=(((1,), (1,)), ((), ())),
        preferred_element_type=jnp.float32,
    )

    lane = jax.lax.broadcasted_iota(jnp.int32, (T, E), 1)
    work = logits
    sel = jnp.zeros((T, E), dtype=jnp.bool_)
    vals = []
    idxs = []
    for _ in range(K):
        m = jnp.max(work, axis=1, keepdims=True)
        is_max = work == m
        j = jnp.min(jnp.where(is_max, lane, E), axis=1, keepdims=True)
        chosen = lane == j
        vals.append(m)
        idxs.append(j)
        sel = jnp.logical_or(sel, chosen)
        work = jnp.where(chosen, -jnp.inf, work)

    top_vals = jnp.concatenate(vals, axis=1)
    top_idx = jnp.concatenate(idxs, axis=1)
    mx = top_vals[:, 0:1]
    ex = jnp.exp(top_vals - mx)
    denom = jnp.sum(ex, axis=1, keepdims=True)
    wts_ref[...] = ex / denom
    idx_ref[...] = top_idx
    scores_ref[...] = jnp.where(sel, jnp.exp(logits - mx) / denom, 0.0)


def kernel(x, W):
    B, S, H = x.shape
    E = W.shape[0]
    K = 8
    N = B * S
    T = 256
    while N % T:
        T //= 2
    xr = x.reshape(N, H)
    grid = (N // T,)
    scores, idx, wts = pl.pallas_call(
        functools.partial(_fused_body, K=K),
        grid=grid,
        in_specs=[
            pl.BlockSpec((T, H), lambda i: (i, 0)),
            pl.BlockSpec((E, H), lambda i: (0, 0)),
        ],
        out_specs=[
            pl.BlockSpec((T, E), lambda i: (i, 0)),
            pl.BlockSpec((T, K), lambda i: (i, 0)),
            pl.BlockSpec((T, K), lambda i: (i, 0)),
        ],
        out_shape=[
            jax.ShapeDtypeStruct((N, E), jnp.float32),
            jax.ShapeDtypeStruct((N, K), jnp.int32),
            jax.ShapeDtypeStruct((N, K), jnp.float32),
        ],
    )(xr, W)
    return (scores.reshape(B, S, E), idx.reshape(B, S, K), wts.reshape(B, S, K))


# speedup vs baseline: 4.0772x; 1.2998x over previous
import functools

import jax
import jax.numpy as jnp
from jax.experimental import pallas as pl
from jax.experimental.pallas import tpu as pltpu


def _fused_body(x_ref, w_ref, scores_ref, idx_ref, wts_ref, *, K):
    T = x_ref.shape[0]
    E = w_ref.shape[1]
    logits = jax.lax.dot_general(
        x_ref[...], w_ref[...],
        dimension_numbers=(((1,), (0,)), ((), ())),
        preferred_element_type=jnp.float32,
    )

    lane = jax.lax.broadcasted_iota(jnp.int32, (T, E), 1)
    work = logits
    sel = jnp.zeros((T, E), dtype=jnp.bool_)
    vals = []
    idxs = []
    for _ in range(K):
        m = jnp.max(work, axis=1, keepdims=True)
        is_max = work == m
        j = jnp.min(jnp.where(is_max, lane, E), axis=1, keepdims=True)
        chosen = lane == j
        vals.append(m)
        idxs.append(j)
        sel = jnp.logical_or(sel, chosen)
        work = jnp.where(chosen, -jnp.inf, work)

    top_vals = jnp.concatenate(vals, axis=1)
    top_idx = jnp.concatenate(idxs, axis=1)
    mx = top_vals[:, 0:1]
    ex = jnp.exp(top_vals - mx)
    denom = jnp.sum(ex, axis=1, keepdims=True)
    wts_ref[...] = ex / denom
    idx_ref[...] = top_idx
    scores_ref[...] = jnp.where(sel, jnp.exp(logits - mx) / denom, 0.0)


def kernel(x, W):
    B, S, H = x.shape
    E = W.shape[0]
    K = 8
    N = B * S
    T = 512
    while N % T:
        T //= 2
    xr = x.reshape(N, H)
    wt = W.T
    grid = (N // T,)
    scores, idx, wts = pl.pallas_call(
        functools.partial(_fused_body, K=K),
        grid=grid,
        in_specs=[
            pl.BlockSpec((T, H), lambda i: (i, 0)),
            pl.BlockSpec((H, E), lambda i: (0, 0)),
        ],
        out_specs=[
            pl.BlockSpec((T, E), lambda i: (i, 0)),
            pl.BlockSpec((T, K), lambda i: (i, 0)),
            pl.BlockSpec((T, K), lambda i: (i, 0)),
        ],
        out_shape=[
            jax.ShapeDtypeStruct((N, E), jnp.float32),
            jax.ShapeDtypeStruct((N, K), jnp.int32),
            jax.ShapeDtypeStruct((N, K), jnp.float32),
        ],
    )(xr, wt)
    return (scores.reshape(B, S, E), idx.reshape(B, S, K), wts.reshape(B, S, K))


# speedup vs baseline: 4.3483x; 1.0665x over previous
import functools

import jax
import jax.numpy as jnp
from jax.experimental import pallas as pl
from jax.experimental.pallas import tpu as pltpu


def _fused_body(x_ref, w_ref, scores_ref, idx_ref, wts_ref, *, K):
    T = x_ref.shape[0]
    E = w_ref.shape[1]
    logits = jax.lax.dot_general(
        x_ref[...], w_ref[...],
        dimension_numbers=(((1,), (0,)), ((), ())),
        preferred_element_type=jnp.float32,
    )

    lane = jax.lax.broadcasted_iota(jnp.int32, (T, E), 1)
    work = logits
    sel = jnp.zeros((T, E), dtype=jnp.bool_)
    vals = []
    idxs = []
    for _ in range(K):
        m = jnp.max(work, axis=1, keepdims=True)
        is_max = work == m
        j = jnp.min(jnp.where(is_max, lane, E), axis=1, keepdims=True)
        chosen = lane == j
        vals.append(m)
        idxs.append(j)
        sel = jnp.logical_or(sel, chosen)
        work = jnp.where(chosen, -jnp.inf, work)

    top_vals = jnp.concatenate(vals, axis=1)
    top_idx = jnp.concatenate(idxs, axis=1)
    mx = top_vals[:, 0:1]
    ex = jnp.exp(top_vals - mx)
    denom = jnp.sum(ex, axis=1, keepdims=True)
    wts_ref[...] = ex / denom
    idx_ref[...] = top_idx
    scores_ref[...] = jnp.where(sel, jnp.exp(logits - mx) / denom, 0.0)


def kernel(x, W):
    B, S, H = x.shape
    E = W.shape[0]
    K = 8
    N = B * S
    T = 1024
    while N % T:
        T //= 2
    xr = x.reshape(N, H)
    wt = W.T
    grid = (N // T,)
    scores, idx, wts = pl.pallas_call(
        functools.partial(_fused_body, K=K),
        grid=grid,
        in_specs=[
            pl.BlockSpec((T, H), lambda i: (i, 0)),
            pl.BlockSpec((H, E), lambda i: (0, 0)),
        ],
        out_specs=[
            pl.BlockSpec((T, E), lambda i: (i, 0)),
            pl.BlockSpec((T, K), lambda i: (i, 0)),
            pl.BlockSpec((T, K), lambda i: (i, 0)),
        ],
        out_shape=[
            jax.ShapeDtypeStruct((N, E), jnp.float32),
            jax.ShapeDtypeStruct((N, K), jnp.int32),
            jax.ShapeDtypeStruct((N, K), jnp.float32),
        ],
    )(xr, wt)
    return (scores.reshape(B, S, E), idx.reshape(B, S, K), wts.reshape(B, S, K))
